# baseline (speedup 1.0000x reference)
import jax
import jax.numpy as jnp
from jax.experimental import pallas as pl
from jax.experimental.pallas import tpu as pltpu

_BN = 2000


def _fused_kernel(x_ref, wz_ref, bz_ref, wh_ref, bh_ref, wlin_ref, blin_ref,
                  out_ref):
    f_in = x_ref.shape[1]
    wz = 0.5 * (wz_ref[0, 0] + wz_ref[1, 0])
    wh = wh_ref[0, 0] + wh_ref[1, 0]
    wcat = jnp.concatenate([wz[:f_in], wh[:f_in]], axis=1)
    bcat = jnp.concatenate([0.5 * bz_ref[...], bh_ref[...]], axis=0)
    x = x_ref[...]
    y = jnp.dot(x, wcat, preferred_element_type=jnp.float32)
    t = jnp.tanh(y + bcat)
    f_out = wz.shape[1]
    h = jnp.maximum(0.5 * (1.0 - t[:, :f_out]) * t[:, f_out:], 0.0)
    out_ref[...] = (jnp.dot(h, wlin_ref[...],
                            preferred_element_type=jnp.float32)
                    + blin_ref[...])


def kernel(x, edge_index, edge_weight, Wz, bz, Wr, br, Wh, bh, W_lin, b_lin):
    del edge_index, edge_weight, Wr, br
    n, f_in = x.shape
    f_out = Wz.shape[-1]
    grid = n // _BN
    out = pl.pallas_call(
        _fused_kernel,
        grid=(grid,),
        in_specs=[
            pl.BlockSpec((_BN, f_in), lambda i: (i, 0)),
            pl.BlockSpec(Wz.shape, lambda i: (0, 0, 0, 0)),
            pl.BlockSpec((f_out,), lambda i: (0,)),
            pl.BlockSpec(Wh.shape, lambda i: (0, 0, 0, 0)),
            pl.BlockSpec((f_out,), lambda i: (0,)),
            pl.BlockSpec((f_out, 1), lambda i: (0, 0)),
            pl.BlockSpec((1,), lambda i: (0,)),
        ],
        out_specs=pl.BlockSpec((_BN, 1), lambda i: (i, 0)),
        out_shape=jax.ShapeDtypeStruct((n, 1), x.dtype),
        compiler_params=pltpu.CompilerParams(
            dimension_semantics=("parallel",)),
    )(x, Wz, bz, Wh, bh, W_lin, b_lin)
    return out


# speedup vs baseline: 1.0896x; 1.0896x over previous
import jax
import jax.numpy as jnp
from jax.experimental import pallas as pl
from jax.experimental.pallas import tpu as pltpu

_BN = 10000


def _fused_kernel(x_ref, wz_ref, bz_ref, wh_ref, bh_ref, wlin_ref, blin_ref,
                  out_ref):
    f_in = x_ref.shape[1]
    wz = 0.5 * (wz_ref[0, 0] + wz_ref[1, 0])
    wh = wh_ref[0, 0] + wh_ref[1, 0]
    wcat = jnp.concatenate([wz[:f_in], wh[:f_in]], axis=1)
    bcat = jnp.concatenate([0.5 * bz_ref[...], bh_ref[...]], axis=0)
    x = x_ref[...]
    y = jnp.dot(x, wcat, preferred_element_type=jnp.float32)
    t = jnp.tanh(y + bcat)
    f_out = wz.shape[1]
    h = jnp.maximum(0.5 * (1.0 - t[:, :f_out]) * t[:, f_out:], 0.0)
    out_ref[...] = (jnp.dot(h, wlin_ref[...],
                            preferred_element_type=jnp.float32)
                    + blin_ref[...])


def kernel(x, edge_index, edge_weight, Wz, bz, Wr, br, Wh, bh, W_lin, b_lin):
    del edge_index, edge_weight, Wr, br
    n, f_in = x.shape
    f_out = Wz.shape[-1]
    grid = n // _BN
    out = pl.pallas_call(
        _fused_kernel,
        grid=(grid,),
        in_specs=[
            pl.BlockSpec((_BN, f_in), lambda i: (i, 0)),
            pl.BlockSpec(Wz.shape, lambda i: (0, 0, 0, 0)),
            pl.BlockSpec((f_out,), lambda i: (0,)),
            pl.BlockSpec(Wh.shape, lambda i: (0, 0, 0, 0)),
            pl.BlockSpec((f_out,), lambda i: (0,)),
            pl.BlockSpec((f_out, 1), lambda i: (0, 0)),
            pl.BlockSpec((1,), lambda i: (0,)),
        ],
        out_specs=pl.BlockSpec((_BN, 1), lambda i: (i, 0)),
        out_shape=jax.ShapeDtypeStruct((n, 1), x.dtype),
        compiler_params=pltpu.CompilerParams(
            dimension_semantics=("parallel",)),
    )(x, Wz, bz, Wh, bh, W_lin, b_lin)
    return out
